# baseline (speedup 1.0000x reference)
import functools

import jax
import jax.numpy as jnp
from jax import lax
from jax.experimental import pallas as pl
from jax.experimental.pallas import tpu as pltpu
from jax.experimental.pallas import tpu_sc as plsc

_LANES = 16
_NW = 32
_NC = 2


def _make_scan_kernel(n_edges, node_pad):
    epw = n_edges // _NW
    nvregs = epw // _LANES
    mesh = plsc.VectorSubcoreMesh(core_axis_name="c", subcore_axis_name="s")

    @functools.partial(
        pl.kernel,
        out_type=jax.ShapeDtypeStruct((_NW * node_pad,), jnp.int32),
        mesh=mesh,
        compiler_params=pltpu.CompilerParams(needs_layout_passes=False),
        scratch_types=[
            pltpu.VMEM((epw,), jnp.int32),
            pltpu.VMEM((node_pad,), jnp.int32),
        ],
    )
    def scan_kernel(pk_hbm, tabs_hbm, pkv, tab):
        wid = lax.axis_index("s") * _NC + lax.axis_index("c")
        base = wid * epw
        pltpu.sync_copy(pk_hbm.at[pl.ds(base, epw)], pkv)

        zero16 = jnp.zeros((_LANES,), jnp.int32)

        init_unroll = 8
        def init_body(i, carry):
            for u in range(init_unroll):
                tab[pl.ds((i * init_unroll + u) * _LANES, _LANES)] = zero16
            return carry

        lax.fori_loop(0, node_pad // _LANES // init_unroll, init_body, 0)

        wtag = (wid + 1) << 14

        def one_vreg(off):
            pk = pkv[pl.ds(off, _LANES)]
            n = pk >> 14
            h = pk & 0x3FFF
            pval = wtag | h
            plsc.store_scatter(tab, [n], pval)

        unroll = 5
        def body(i, carry):
            for u in range(unroll):
                one_vreg((i * unroll + u) * _LANES)
            return carry

        lax.fori_loop(0, nvregs // unroll, body, 0)
        for r in range(nvregs - (nvregs // unroll) * unroll):
            one_vreg(((nvregs // unroll) * unroll + r) * _LANES)
        pltpu.sync_copy(tab, tabs_hbm.at[pl.ds(wid * node_pad, node_pad)])

    return scan_kernel


def _make_combine_gather_kernel(node_pad, n_hyper, dim):
    npw = node_pad // _NW
    n_chunks = npw // 128
    ns = _NW // _NC
    rows_per_s = (n_hyper // ns) & ~7
    tail = n_hyper - rows_per_s * ns
    mesh = plsc.VectorSubcoreMesh(core_axis_name="c", subcore_axis_name="s")

    @functools.partial(
        pl.kernel,
        out_type=jax.ShapeDtypeStruct((node_pad, dim), jnp.float32),
        mesh=mesh,
        compiler_params=pltpu.CompilerParams(needs_layout_passes=False),
        scratch_types=[
            pltpu.VMEM((_NW, npw), jnp.int32),
            pltpu.VMEM((n_chunks, 128), jnp.int32),
            pltpu.VMEM((2, 128, dim), jnp.float32),
            pltpu.VMEM_SHARED((n_hyper + 8, dim), jnp.float32),
            pltpu.SemaphoreType.DMA,
            pltpu.SemaphoreType.DMA,
            pltpu.SemaphoreType.DMA,
            pltpu.SemaphoreType.DMA,
        ],
    )
    def combine_kernel(tabs_hbm, hyper_hbm, nan_hbm, h_rows_hbm, tabv, hsel, rows, hyp_sp, sem, sem_st, sg0, sg1):
        sid = lax.axis_index("s")
        wid = sid * _NC + lax.axis_index("c")
        nbase = wid * npw
        stage = pltpu.make_async_copy(
            hyper_hbm.at[pl.ds(sid * rows_per_s, rows_per_s)],
            hyp_sp.at[pl.ds(sid * rows_per_s, rows_per_s)], sem_st)
        stage.start()
        tail_stage = pltpu.make_async_copy(
            hyper_hbm.at[pl.ds(ns * rows_per_s, tail)],
            hyp_sp.at[pl.ds(ns * rows_per_s, tail)], sem_st)
        nan_stage = pltpu.make_async_copy(
            nan_hbm, hyp_sp.at[pl.ds(n_hyper, 8)], sem_st)
        @pl.when(sid == 0)
        def _():
            tail_stage.start()
            nan_stage.start()
        cps = [pltpu.make_async_copy(
                   tabs_hbm.at[pl.ds(l * node_pad + nbase, npw)],
                   tabv.at[l], sem) for l in range(_NW)]
        for cp in cps:
            cp.start()
        for cp in cps:
            cp.wait()

        sentinel = jnp.zeros((_LANES,), jnp.int32) + (n_hyper + (wid & 7))
        for j in range(npw // _LANES):
            sl = pl.ds(j * _LANES, _LANES)
            acc = tabv[0, sl]
            for l in range(1, _NW):
                acc = jnp.maximum(acc, tabv[l, sl])
            h = jnp.where(acc > 0, acc & 0x3FFF, sentinel)
            pos = j * _LANES
            hsel[pos // 128, pl.ds(pos % 128, _LANES)] = h

        stage.wait()
        @pl.when(sid == 0)
        def _():
            tail_stage.wait()
            nan_stage.wait()
        plsc.subcore_barrier()

        sems = (sg0, sg1)
        pending = [None, None]
        for c in range(n_chunks):
            b = c % 2
            pending[b] = (c, pltpu.async_copy(
                hyp_sp.at[hsel.at[c]], rows.at[b], sems[b]))
            if c >= 1:
                wc, wcp = pending[(c - 1) % 2]
                wcp.wait()
                pltpu.sync_copy(rows.at[(c - 1) % 2],
                                h_rows_hbm.at[pl.ds(nbase + wc * 128, 128)])
                pending[(c - 1) % 2] = None
        lastc, lastcp = pending[(n_chunks - 1) % 2]
        lastcp.wait()
        pltpu.sync_copy(rows.at[(n_chunks - 1) % 2],
                        h_rows_hbm.at[pl.ds(nbase + lastc * 128, 128)])

    return combine_kernel


def _tc_matmul_body(h_ref, node_ref, wt_ref, wb_ref, out_ref):
    hh = h_ref[...]
    acc = jnp.dot(hh, wt_ref[...], preferred_element_type=jnp.float32)
    acc = acc + jnp.dot(node_ref[...], wb_ref[...], preferred_element_type=jnp.float32)
    c0 = hh[:, 0:1]
    out_ref[...] = jnp.where(c0 == c0, acc, 0.0)


def kernel(node_embeddings, semalink_embeddings, hyperedge_embeddings, semalinks, W_a, W_f):
    del semalink_embeddings, W_a
    n_nodes, in_dim = node_embeddings.shape
    n_edges = semalinks.shape[0]
    out_dim = W_f.shape[1]
    unit = _NW * 128
    node_pad = ((n_nodes + unit - 1) // unit) * unit

    sl = semalinks.astype(jnp.int32)
    pk = (sl[:, 0] << 14) | sl[:, 2]
    nan_rows = jnp.full((8, in_dim), jnp.nan, dtype=jnp.float32)

    tabs = _make_scan_kernel(n_edges, node_pad)(pk)
    h_rows = _make_combine_gather_kernel(
        node_pad, hyperedge_embeddings.shape[0], in_dim)(
        tabs, hyperedge_embeddings, nan_rows)

    wt = W_f[:in_dim]
    wb = W_f[in_dim:]

    blk = 1000
    grid = (n_nodes // blk,)
    out = pl.pallas_call(
        _tc_matmul_body,
        grid=grid,
        in_specs=[
            pl.BlockSpec((blk, in_dim), lambda i: (i, 0)),
            pl.BlockSpec((blk, in_dim), lambda i: (i, 0)),
            pl.BlockSpec((in_dim, out_dim), lambda i: (0, 0)),
            pl.BlockSpec((in_dim, out_dim), lambda i: (0, 0)),
        ],
        out_specs=pl.BlockSpec((blk, out_dim), lambda i: (i, 0)),
        out_shape=jax.ShapeDtypeStruct((n_nodes, out_dim), jnp.float32),
    )(h_rows, node_embeddings, wt, wb)
    return out


# speedup vs baseline: 1.0587x; 1.0587x over previous
import functools

import jax
import jax.numpy as jnp
from jax import lax
from jax.experimental import pallas as pl
from jax.experimental.pallas import tpu as pltpu
from jax.experimental.pallas import tpu_sc as plsc

_LANES = 16
_NW = 32
_NC = 2


def _make_scan_kernel(n_edges, node_pad):
    epw = n_edges // _NW
    nvregs = epw // _LANES
    mesh = plsc.VectorSubcoreMesh(core_axis_name="c", subcore_axis_name="s")

    @functools.partial(
        pl.kernel,
        out_type=jax.ShapeDtypeStruct((_NW * node_pad,), jnp.int32),
        mesh=mesh,
        compiler_params=pltpu.CompilerParams(needs_layout_passes=False),
        scratch_types=[
            pltpu.VMEM((epw,), jnp.int32),
            pltpu.VMEM((epw,), jnp.int32),
            pltpu.VMEM((node_pad,), jnp.int32),
        ],
    )
    def scan_kernel(nidx_hbm, hidx_hbm, tabs_hbm, nv, hv, tab):
        wid = lax.axis_index("s") * _NC + lax.axis_index("c")
        base = wid * epw
        pltpu.sync_copy(nidx_hbm.at[pl.ds(base, epw)], nv)
        pltpu.sync_copy(hidx_hbm.at[pl.ds(base, epw)], hv)

        zero16 = jnp.zeros((_LANES,), jnp.int32)

        init_unroll = 8
        def init_body(i, carry):
            for u in range(init_unroll):
                tab[pl.ds((i * init_unroll + u) * _LANES, _LANES)] = zero16
            return carry

        lax.fori_loop(0, node_pad // _LANES // init_unroll, init_body, 0)

        wtag = (wid + 1) << 14

        def one_vreg(off):
            n = nv[pl.ds(off, _LANES)]
            h = hv[pl.ds(off, _LANES)]
            pval = wtag | h
            plsc.store_scatter(tab, [n], pval)

        unroll = 5
        def body(i, carry):
            for u in range(unroll):
                one_vreg((i * unroll + u) * _LANES)
            return carry

        lax.fori_loop(0, nvregs // unroll, body, 0)
        for r in range(nvregs - (nvregs // unroll) * unroll):
            one_vreg(((nvregs // unroll) * unroll + r) * _LANES)
        pltpu.sync_copy(tab, tabs_hbm.at[pl.ds(wid * node_pad, node_pad)])

    return scan_kernel


def _make_combine_gather_kernel(node_pad, n_hyper, dim):
    npw = node_pad // _NW
    n_chunks = npw // 128
    ns = _NW // _NC
    rows_per_s = (n_hyper // ns) & ~7
    tail = n_hyper - rows_per_s * ns
    mesh = plsc.VectorSubcoreMesh(core_axis_name="c", subcore_axis_name="s")

    @functools.partial(
        pl.kernel,
        out_type=jax.ShapeDtypeStruct((node_pad, dim), jnp.float32),
        mesh=mesh,
        compiler_params=pltpu.CompilerParams(needs_layout_passes=False),
        scratch_types=[
            pltpu.VMEM((_NW, npw), jnp.int32),
            pltpu.VMEM((n_chunks, 128), jnp.int32),
            pltpu.VMEM((2, 128, dim), jnp.float32),
            pltpu.VMEM_SHARED((n_hyper + 8, dim), jnp.float32),
            pltpu.SemaphoreType.DMA,
            pltpu.SemaphoreType.DMA,
            pltpu.SemaphoreType.DMA,
            pltpu.SemaphoreType.DMA,
        ],
    )
    def combine_kernel(tabs_hbm, hyper_hbm, nan_hbm, h_rows_hbm, tabv, hsel, rows, hyp_sp, sem, sem_st, sg0, sg1):
        sid = lax.axis_index("s")
        wid = sid * _NC + lax.axis_index("c")
        nbase = wid * npw
        stage = pltpu.make_async_copy(
            hyper_hbm.at[pl.ds(sid * rows_per_s, rows_per_s)],
            hyp_sp.at[pl.ds(sid * rows_per_s, rows_per_s)], sem_st)
        stage.start()
        tail_stage = pltpu.make_async_copy(
            hyper_hbm.at[pl.ds(ns * rows_per_s, tail)],
            hyp_sp.at[pl.ds(ns * rows_per_s, tail)], sem_st)
        nan_stage = pltpu.make_async_copy(
            nan_hbm, hyp_sp.at[pl.ds(n_hyper, 8)], sem_st)
        @pl.when(sid == 0)
        def _():
            tail_stage.start()
            nan_stage.start()
        cps = [pltpu.make_async_copy(
                   tabs_hbm.at[pl.ds(l * node_pad + nbase, npw)],
                   tabv.at[l], sem) for l in range(_NW)]
        for cp in cps:
            cp.start()
        for cp in cps:
            cp.wait()

        sentinel = jnp.zeros((_LANES,), jnp.int32) + (n_hyper + (wid & 7))
        for j in range(npw // _LANES):
            sl = pl.ds(j * _LANES, _LANES)
            acc = tabv[0, sl]
            for l in range(1, _NW):
                acc = jnp.maximum(acc, tabv[l, sl])
            h = jnp.where(acc > 0, acc & 0x3FFF, sentinel)
            pos = j * _LANES
            hsel[pos // 128, pl.ds(pos % 128, _LANES)] = h

        stage.wait()
        @pl.when(sid == 0)
        def _():
            tail_stage.wait()
            nan_stage.wait()
        plsc.subcore_barrier()

        sems = (sg0, sg1)
        pending = [None, None]
        for c in range(n_chunks):
            b = c % 2
            pending[b] = (c, pltpu.async_copy(
                hyp_sp.at[hsel.at[c]], rows.at[b], sems[b]))
            if c >= 1:
                wc, wcp = pending[(c - 1) % 2]
                wcp.wait()
                pltpu.sync_copy(rows.at[(c - 1) % 2],
                                h_rows_hbm.at[pl.ds(nbase + wc * 128, 128)])
                pending[(c - 1) % 2] = None
        lastc, lastcp = pending[(n_chunks - 1) % 2]
        lastcp.wait()
        pltpu.sync_copy(rows.at[(n_chunks - 1) % 2],
                        h_rows_hbm.at[pl.ds(nbase + lastc * 128, 128)])

    return combine_kernel


def _tc_matmul_body(h_ref, node_ref, wt_ref, wb_ref, out_ref):
    hh = h_ref[...]
    acc = jnp.dot(hh, wt_ref[...], preferred_element_type=jnp.float32)
    acc = acc + jnp.dot(node_ref[...], wb_ref[...], preferred_element_type=jnp.float32)
    c0 = hh[:, 0:1]
    out_ref[...] = jnp.where(c0 == c0, acc, 0.0)


def kernel(node_embeddings, semalink_embeddings, hyperedge_embeddings, semalinks, W_a, W_f):
    del semalink_embeddings, W_a
    n_nodes, in_dim = node_embeddings.shape
    n_edges = semalinks.shape[0]
    out_dim = W_f.shape[1]
    unit = _NW * 128
    node_pad = ((n_nodes + unit - 1) // unit) * unit

    sl = semalinks.astype(jnp.int32)
    n_idx = sl[:, 0]
    h_idx = sl[:, 2]
    nan_rows = jnp.full((8, in_dim), jnp.nan, dtype=jnp.float32)

    tabs = _make_scan_kernel(n_edges, node_pad)(n_idx, h_idx)
    h_rows = _make_combine_gather_kernel(
        node_pad, hyperedge_embeddings.shape[0], in_dim)(
        tabs, hyperedge_embeddings, nan_rows)

    wt = W_f[:in_dim]
    wb = W_f[in_dim:]

    blk = 2000
    grid = (n_nodes // blk,)
    out = pl.pallas_call(
        _tc_matmul_body,
        grid=grid,
        in_specs=[
            pl.BlockSpec((blk, in_dim), lambda i: (i, 0)),
            pl.BlockSpec((blk, in_dim), lambda i: (i, 0)),
            pl.BlockSpec((in_dim, out_dim), lambda i: (0, 0)),
            pl.BlockSpec((in_dim, out_dim), lambda i: (0, 0)),
        ],
        out_specs=pl.BlockSpec((blk, out_dim), lambda i: (i, 0)),
        out_shape=jax.ShapeDtypeStruct((n_nodes, out_dim), jnp.float32),
    )(h_rows, node_embeddings, wt, wb)
    return out
